# baseline (speedup 1.0000x reference)
import functools

import jax
import jax.numpy as jnp
from jax import lax
from jax.experimental import pallas as pl
from jax.experimental.pallas import tpu as pltpu
from jax.experimental.pallas import tpu_sc as plsc

NC = 2
NS = 16
LANES = 16
CHUNK = 128
NBUF = 3
SLOW_FRAC = 42


def _sc_spmm(xmat, edges, nch0, nch1):
    n, d = xmat.shape
    nvreg = d // LANES
    mesh = plsc.VectorSubcoreMesh(
        core_axis_name="c", subcore_axis_name="s",
        num_cores=NC, num_subcores=NS)

    @functools.partial(
        pl.kernel, mesh=mesh,
        out_type=jax.ShapeDtypeStruct((NC, n, d), jnp.float32),
        scratch_types=[
            pltpu.VMEM((NBUF, 3, CHUNK), jnp.int32),
            pltpu.VMEM((NBUF, CHUNK, d), jnp.float32),
            pltpu.VMEM_SHARED((n, d), jnp.float32),
        ] + [pltpu.SemaphoreType.DMA] * (2 * NBUF),
    )
    def spmm(x_hbm, e_hbm, out_hbm, ebuf, rows, acc, *sems):
        gsems, ssems = sems[:NBUF], sems[NBUF:]
        c = lax.axis_index("c")
        s = lax.axis_index("s")

        zval = jnp.zeros((LANES,), jnp.float32)

        def zbody(i, _):
            for f in range(nvreg):
                rows[0, i, pl.ds(f * LANES, LANES)] = zval
            return 0

        lax.fori_loop(0, CHUNK, zbody, 0)

        nrk = -(-n // CHUNK)
        for k in range(nrk):
            size = min(CHUNK, n - k * CHUNK)

            def _zero(off=k * CHUNK, sz=size):
                pltpu.sync_copy(rows.at[0, pl.ds(0, sz)],
                                acc.at[pl.ds(off, sz)])

            pl.when(jnp.equal(k % NS, s))(_zero)

        nch = jnp.where(jnp.equal(c, 0), nch0, nch1)
        ebase = jnp.where(jnp.equal(c, 0),
                          s * nch0, NS * nch0 + s * nch1) * CHUNK

        def load_and_gather(g, b):
            pltpu.sync_copy(e_hbm.at[:, pl.ds(ebase + g * CHUNK, CHUNK)],
                            ebuf.at[b])
            pltpu.async_copy(x_hbm.at[ebuf.at[b, 0]], rows.at[b], gsems[b])

        for g0 in range(2):
            load_and_gather(g0, g0)
        plsc.subcore_barrier()

        def ring_body(gp, _):
            for b in range(NBUF):
                g = gp * NBUF + b

                pltpu.make_async_copy(
                    x_hbm.at[ebuf.at[b, 0]], rows.at[b], gsems[b]).wait()

                def scale_body(q, _):
                    wq = ebuf[b, 2, pl.ds(q * LANES, LANES)]
                    w16 = wq.astype(jnp.float32) * (2.0 ** -20)
                    for e in range(LANES):
                        we = w16[e]
                        row = q * LANES + e
                        for f in range(nvreg):
                            sl = pl.ds(f * LANES, LANES)
                            rows[b, row, sl] = rows[b, row, sl] * we
                    return 0

                lax.fori_loop(0, CHUNK // LANES, scale_body, 0)

                pltpu.async_copy(rows.at[b], acc.at[ebuf.at[b, 1]], ssems[b],
                                 add=True)

                bn = (b + 2) % NBUF

                def wait_prev(bb=bn):
                    pltpu.make_async_copy(
                        rows.at[bb], acc.at[ebuf.at[bb, 1]], ssems[bb]).wait()

                pl.when(g >= 1)(wait_prev)

                def fire_next(bb=bn):
                    load_and_gather(g + 2, bb)

                pl.when(g + 2 < nch)(fire_next)
            return 0

        lax.fori_loop(0, nch // NBUF, ring_body, 0)

        bl = (-1) % NBUF
        pltpu.make_async_copy(
            rows.at[bl], acc.at[ebuf.at[bl, 1]], ssems[bl]).wait()

        plsc.subcore_barrier()

        for k in range(nrk):
            size = min(CHUNK, n - k * CHUNK)

            def _flush(off=k * CHUNK, sz=size):
                pltpu.sync_copy(acc.at[pl.ds(off, sz)],
                                out_hbm.at[c, pl.ds(off, sz)])

            pl.when(jnp.equal(k % NS, s))(_flush)

    return spmm(xmat, edges)


def _tc_layer(z, yp, Wa, Wb, b, relu, block_n):
    n, d = z.shape
    dout = Wa.shape[1]
    grid = n // block_n

    def body(z_ref, yp_ref, wa_ref, wb_ref, b_ref, o_ref):
        ysum = yp_ref[0] + yp_ref[1]
        acc = jnp.dot(z_ref[...], wa_ref[...],
                      preferred_element_type=jnp.float32)
        acc += jnp.dot(ysum, wb_ref[...], preferred_element_type=jnp.float32)
        acc += b_ref[...][None, :]
        if relu:
            acc = jnp.maximum(acc, 0.0)
        o_ref[...] = acc

    return pl.pallas_call(
        body,
        grid=(grid,),
        in_specs=[
            pl.BlockSpec((block_n, d), lambda i: (i, 0)),
            pl.BlockSpec((2, block_n, d), lambda i: (0, i, 0)),
            pl.BlockSpec((d, dout), lambda i: (0, 0)),
            pl.BlockSpec((d, dout), lambda i: (0, 0)),
            pl.BlockSpec((dout,), lambda i: (0,)),
        ],
        out_specs=pl.BlockSpec((block_n, dout), lambda i: (i, 0)),
        out_shape=jax.ShapeDtypeStruct((n, dout), jnp.float32),
    )(z, yp, Wa, Wb, b)


def kernel(x, edge_index, edge_weight, W1, b1, W2, b2):
    n, d = x.shape
    e = edge_weight.shape[0]

    W1a, W1b = W1[0::2], W1[1::2]
    W2a, W2b = W2[0::2], W2[1::2]

    ct = (-(-e // (NS * CHUNK)) + NBUF - 1) // NBUF * NBUF
    nch1 = max(NBUF, (ct * SLOW_FRAC // 159) // NBUF * NBUF)
    nch0 = ct - nch1
    e_pad = ct * NS * CHUNK
    pad = e_pad - e
    src = jnp.pad(edge_index[1], (0, pad))
    dst = jnp.pad(edge_index[0], (0, pad))
    w_fix = jnp.round(
        jnp.pad(edge_weight, (0, pad)) * (2.0 ** 20)).astype(jnp.int32)
    edges = jnp.stack([src, dst, w_fix])

    y1 = _sc_spmm(x, edges, nch0, nch1)
    h = _tc_layer(x, y1, W1a, W1b, b1, relu=True, block_n=1000)
    y2 = _sc_spmm(h, edges, nch0, nch1)
    out = _tc_layer(h, y2, W2a, W2b, b2, relu=False, block_n=1000)
    return out


# speedup vs baseline: 1.0596x; 1.0596x over previous
import functools

import jax
import jax.numpy as jnp
from jax import lax
from jax.experimental import pallas as pl
from jax.experimental.pallas import tpu as pltpu
from jax.experimental.pallas import tpu_sc as plsc

NC = 2
NS = 16
LANES = 16
CHUNK = 128
NBUF = 3
SLOW_FRAC = 33


def _sc_spmm(xmat, edges, w, nch0, nch1):
    n, d = xmat.shape
    nvreg = d // LANES
    mesh = plsc.VectorSubcoreMesh(
        core_axis_name="c", subcore_axis_name="s",
        num_cores=NC, num_subcores=NS)

    @functools.partial(
        pl.kernel, mesh=mesh,
        out_type=jax.ShapeDtypeStruct((NC, n, d), jnp.float32),
        scratch_types=[
            pltpu.VMEM((NBUF, 2, CHUNK), jnp.int32),
            pltpu.VMEM((NBUF, CHUNK), jnp.float32),
            pltpu.VMEM((NBUF, CHUNK, d), jnp.float32),
            pltpu.VMEM_SHARED((n, d), jnp.float32),
        ] + [pltpu.SemaphoreType.DMA] * (2 * NBUF),
    )
    def spmm(x_hbm, e_hbm, w_hbm, out_hbm, ebuf, wbuf, rows, acc, *sems):
        gsems, ssems = sems[:NBUF], sems[NBUF:]
        c = lax.axis_index("c")
        s = lax.axis_index("s")

        zval = jnp.zeros((LANES,), jnp.float32)

        def zbody(i, _):
            for f in range(nvreg):
                rows[0, i, pl.ds(f * LANES, LANES)] = zval
            return 0

        lax.fori_loop(0, CHUNK, zbody, 0)

        nrk = -(-n // CHUNK)
        for k in range(nrk):
            size = min(CHUNK, n - k * CHUNK)

            def _zero(off=k * CHUNK, sz=size):
                pltpu.sync_copy(rows.at[0, pl.ds(0, sz)],
                                acc.at[pl.ds(off, sz)])

            pl.when(jnp.equal(k % NS, s))(_zero)

        nch = jnp.where(jnp.equal(c, 0), nch0, nch1)
        ebase = jnp.where(jnp.equal(c, 0),
                          s * nch0, NS * nch0 + s * nch1) * CHUNK

        def load_and_gather(g, b):
            pltpu.sync_copy(e_hbm.at[:, pl.ds(ebase + g * CHUNK, CHUNK)],
                            ebuf.at[b])
            pltpu.sync_copy(w_hbm.at[pl.ds(ebase + g * CHUNK, CHUNK)],
                            wbuf.at[b])
            pltpu.async_copy(x_hbm.at[ebuf.at[b, 0]], rows.at[b], gsems[b])

        for g0 in range(2):
            load_and_gather(g0, g0)
        plsc.subcore_barrier()

        def ring_body(gp, _):
            for b in range(NBUF):
                g = gp * NBUF + b

                pltpu.make_async_copy(
                    x_hbm.at[ebuf.at[b, 0]], rows.at[b], gsems[b]).wait()

                def scale_body(q, _):
                    w16 = wbuf[b, pl.ds(q * LANES, LANES)]
                    for e in range(LANES):
                        we = w16[e]
                        row = q * LANES + e
                        for f in range(nvreg):
                            sl = pl.ds(f * LANES, LANES)
                            rows[b, row, sl] = rows[b, row, sl] * we
                    return 0

                lax.fori_loop(0, CHUNK // LANES, scale_body, 0)

                pltpu.async_copy(rows.at[b], acc.at[ebuf.at[b, 1]], ssems[b],
                                 add=True)

                bn = (b + 2) % NBUF

                def wait_prev(bb=bn):
                    pltpu.make_async_copy(
                        rows.at[bb], acc.at[ebuf.at[bb, 1]], ssems[bb]).wait()

                pl.when(g >= 1)(wait_prev)

                def fire_next(bb=bn):
                    load_and_gather(g + 2, bb)

                pl.when(g + 2 < nch)(fire_next)
            return 0

        lax.fori_loop(0, nch // NBUF, ring_body, 0)

        bl = (-1) % NBUF
        pltpu.make_async_copy(
            rows.at[bl], acc.at[ebuf.at[bl, 1]], ssems[bl]).wait()

        plsc.subcore_barrier()

        for k in range(nrk):
            size = min(CHUNK, n - k * CHUNK)

            def _flush(off=k * CHUNK, sz=size):
                pltpu.sync_copy(acc.at[pl.ds(off, sz)],
                                out_hbm.at[c, pl.ds(off, sz)])

            pl.when(jnp.equal(k % NS, s))(_flush)

    return spmm(xmat, edges, w)


def _tc_layer(z, yp, Wa, Wb, b, relu, block_n):
    n, d = z.shape
    dout = Wa.shape[1]
    grid = n // block_n

    def body(z_ref, yp_ref, wa_ref, wb_ref, b_ref, o_ref):
        ysum = yp_ref[0] + yp_ref[1]
        acc = jnp.dot(z_ref[...], wa_ref[...],
                      preferred_element_type=jnp.float32)
        acc += jnp.dot(ysum, wb_ref[...], preferred_element_type=jnp.float32)
        acc += b_ref[...][None, :]
        if relu:
            acc = jnp.maximum(acc, 0.0)
        o_ref[...] = acc

    return pl.pallas_call(
        body,
        grid=(grid,),
        in_specs=[
            pl.BlockSpec((block_n, d), lambda i: (i, 0)),
            pl.BlockSpec((2, block_n, d), lambda i: (0, i, 0)),
            pl.BlockSpec((d, dout), lambda i: (0, 0)),
            pl.BlockSpec((d, dout), lambda i: (0, 0)),
            pl.BlockSpec((dout,), lambda i: (0,)),
        ],
        out_specs=pl.BlockSpec((block_n, dout), lambda i: (i, 0)),
        out_shape=jax.ShapeDtypeStruct((n, dout), jnp.float32),
    )(z, yp, Wa, Wb, b)


def kernel(x, edge_index, edge_weight, W1, b1, W2, b2):
    n, d = x.shape
    e = edge_weight.shape[0]

    W1a, W1b = W1[0::2], W1[1::2]
    W2a, W2b = W2[0::2], W2[1::2]

    ct = (-(-e // (NS * CHUNK)) + NBUF - 1) // NBUF * NBUF
    nch1 = max(NBUF, (ct * SLOW_FRAC // 159) // NBUF * NBUF)
    nch0 = ct - nch1
    e_pad = ct * NS * CHUNK
    pad = e_pad - e
    src = jnp.pad(edge_index[1], (0, pad))
    dst = jnp.pad(edge_index[0], (0, pad))
    w = jnp.pad(edge_weight, (0, pad))
    edges = jnp.stack([src, dst])

    y1 = _sc_spmm(x, edges, w, nch0, nch1)
    h = _tc_layer(x, y1, W1a, W1b, b1, relu=True, block_n=1000)
    y2 = _sc_spmm(h, edges, w, nch0, nch1)
    out = _tc_layer(h, y2, W2a, W2b, b2, relu=False, block_n=1000)
    return out
